# baseline (speedup 1.0000x reference)
import jax
import jax.numpy as jnp
import numpy as np
from jax import lax
from jax.experimental import pallas as pl
from jax.experimental.pallas import tpu as pltpu
from jax.experimental.pallas import tpu_sc as plsc

_B, _C, _H, _W = 32, 512, 64, 64
_HW = _H * _W
_MID = max(16, _C // 16)
_NE = 64
_K = 2
_NC, _NS, _L = 2, 16, 16
_CHC = 4
_NCHUNK = _C // _CHC


def _sc_pool_body(x_hbm, out_hbm, buf0, buf1, sums_v, sem0, sem1):
    b = lax.axis_index("s") * _NC + lax.axis_index("c")

    def start(k, buf, sem):
        pltpu.make_async_copy(x_hbm.at[b, pl.ds(k * _CHC, _CHC)],
                              buf, sem).start()

    def wait(buf, sem):
        pltpu.make_async_copy(x_hbm.at[0, pl.ds(0, _CHC)], buf, sem).wait()

    start(0, buf0, sem0)
    start(1, buf1, sem1)

    def chunk_sums(buf, k):
        for ch in range(_CHC):
            def body(r, acc):
                a0 = acc + buf[ch, r, pl.ds(0, _L)]
                a1 = a0 + buf[ch, r, pl.ds(_L, _L)]
                a2 = a1 + buf[ch, r, pl.ds(2 * _L, _L)]
                return a2 + buf[ch, r, pl.ds(3 * _L, _L)]
            acc = lax.fori_loop(0, _H, body, jnp.zeros((_L,), jnp.float32),
                                unroll=4)
            sums_v[pl.ds((k * _CHC + ch) * _L, _L)] = acc

    def loop_body(g, carry):
        wait(buf0, sem0)
        chunk_sums(buf0, 2 * g)

        @pl.when(2 * g + 2 < _NCHUNK)
        def _():
            start(2 * g + 2, buf0, sem0)

        wait(buf1, sem1)
        chunk_sums(buf1, 2 * g + 1)

        @pl.when(2 * g + 3 < _NCHUNK)
        def _():
            start(2 * g + 3, buf1, sem1)
        return carry

    lax.fori_loop(0, _NCHUNK // 2, loop_body, 0)
    pltpu.sync_copy(sums_v, out_hbm.at[b])


def _sc_pool(x):
    mesh = plsc.VectorSubcoreMesh(core_axis_name="c", subcore_axis_name="s")
    kfn = pl.kernel(
        _sc_pool_body,
        mesh=mesh,
        out_type=jax.ShapeDtypeStruct((_B, _C * _L), jnp.float32),
        scratch_types=[
            pltpu.VMEM((_CHC, _H, _W), jnp.float32),
            pltpu.VMEM((_CHC, _H, _W), jnp.float32),
            pltpu.VMEM((_C * _L,), jnp.float32),
            pltpu.SemaphoreType.DMA,
            pltpu.SemaphoreType.DMA,
        ],
    )
    return kfn(x)


def _router_tc_body(p_ref, s_ref, w1_ref, b1_ref, w2_ref, b2_ref,
                    wout_ref, iout_ref):
    g = jnp.dot(p_ref[...], s_ref[...],
                preferred_element_type=jnp.float32,
                precision=lax.Precision.HIGHEST) * (1.0 / _HW)
    h = b1_ref[...] + jnp.dot(g, w1_ref[...],
                              preferred_element_type=jnp.float32)
    h = h * jax.nn.sigmoid(h)
    logits = jnp.dot(h, w2_ref[...],
                     preferred_element_type=jnp.float32) + b2_ref[...]

    idx = lax.broadcasted_iota(jnp.int32, (_B, _NE), 1)
    m1 = jnp.max(logits, axis=1, keepdims=True)
    i1 = jnp.min(jnp.where(logits == m1, idx, _NE), axis=1, keepdims=True)
    masked = jnp.where(idx == i1, -jnp.inf, logits)
    m2 = jnp.max(masked, axis=1, keepdims=True)
    i2 = jnp.min(jnp.where(masked == m2, idx, _NE), axis=1, keepdims=True)

    e = jnp.exp(m2 - m1)
    denom = 1.0 + e
    wout_ref[...] = jnp.concatenate([1.0 / denom, e / denom], axis=1)
    iout_ref[...] = jnp.concatenate([i1, i2], axis=1)


_SEL = np.kron(np.eye(_C, dtype=np.float32), np.ones((_L, 1), np.float32))


def kernel(x, W1, b1, W2, b2):
    partials = _sc_pool(x)
    sel = jnp.asarray(_SEL)
    b1r = b1.reshape(1, _MID)
    b2r = b2.reshape(1, _NE)

    wout, iout = pl.pallas_call(
        _router_tc_body,
        out_shape=[
            jax.ShapeDtypeStruct((_B, _K), jnp.float32),
            jax.ShapeDtypeStruct((_B, _K), jnp.int32),
        ],
    )(partials, sel, W1, b1r, W2, b2r)
    return wout, iout


# speedup vs baseline: 1.0288x; 1.0288x over previous
import jax
import jax.numpy as jnp
import numpy as np
from jax import lax
from jax.experimental import pallas as pl
from jax.experimental.pallas import tpu as pltpu
from jax.experimental.pallas import tpu_sc as plsc

_B, _C, _H, _W = 32, 512, 64, 64
_HW = _H * _W
_MID = max(16, _C // 16)
_NE = 64
_K = 2
_NC, _NS, _L = 2, 16, 16
_CHC = 4
_NCHUNK = _C // _CHC
_NBUF = 3


def _sc_pool_body(x_hbm, out_hbm, buf0, buf1, buf2, sums_v,
                  sem0, sem1, sem2):
    b = lax.axis_index("s") * _NC + lax.axis_index("c")
    bufs = (buf0, buf1, buf2)
    sems = (sem0, sem1, sem2)

    def start(k, j):
        pltpu.make_async_copy(x_hbm.at[b, pl.ds(k * _CHC, _CHC)],
                              bufs[j], sems[j]).start()

    def wait(j):
        pltpu.make_async_copy(x_hbm.at[0, pl.ds(0, _CHC)],
                              bufs[j], sems[j]).wait()

    for j in range(_NBUF):
        start(j, j)

    def chunk_sums(buf, k):
        for ch in range(_CHC):
            def body(r, acc):
                a0 = acc + buf[ch, r, pl.ds(0, _L)]
                a1 = a0 + buf[ch, r, pl.ds(_L, _L)]
                a2 = a1 + buf[ch, r, pl.ds(2 * _L, _L)]
                return a2 + buf[ch, r, pl.ds(3 * _L, _L)]
            acc = lax.fori_loop(0, _H, body, jnp.zeros((_L,), jnp.float32),
                                unroll=8)
            sums_v[pl.ds((k * _CHC + ch) * _L, _L)] = acc

    _NG = _NCHUNK // _NBUF

    def loop_body(g, carry):
        for j in range(_NBUF):
            k = _NBUF * g + j
            wait(j)
            chunk_sums(bufs[j], k)

            @pl.when(k + _NBUF < _NCHUNK)
            def _():
                start(k + _NBUF, j)
        return carry

    lax.fori_loop(0, _NG, loop_body, 0)
    for k in range(_NG * _NBUF, _NCHUNK):
        j = k % _NBUF
        wait(j)
        chunk_sums(bufs[j], k)
    pltpu.sync_copy(sums_v, out_hbm.at[b])


def _sc_pool(x):
    mesh = plsc.VectorSubcoreMesh(core_axis_name="c", subcore_axis_name="s")
    kfn = pl.kernel(
        _sc_pool_body,
        mesh=mesh,
        out_type=jax.ShapeDtypeStruct((_B, _C * _L), jnp.float32),
        scratch_types=[
            pltpu.VMEM((_CHC, _H, _W), jnp.float32),
            pltpu.VMEM((_CHC, _H, _W), jnp.float32),
            pltpu.VMEM((_CHC, _H, _W), jnp.float32),
            pltpu.VMEM((_C * _L,), jnp.float32),
            pltpu.SemaphoreType.DMA,
            pltpu.SemaphoreType.DMA,
            pltpu.SemaphoreType.DMA,
        ],
    )
    return kfn(x)


def _router_tc_body(p_ref, s_ref, w1_ref, b1_ref, w2_ref, b2_ref,
                    wout_ref, iout_ref):
    g = jnp.dot(p_ref[...], s_ref[...],
                preferred_element_type=jnp.float32,
                precision=lax.Precision.HIGHEST) * (1.0 / _HW)
    h = b1_ref[...] + jnp.dot(g, w1_ref[...],
                              preferred_element_type=jnp.float32)
    h = h * jax.nn.sigmoid(h)
    logits = jnp.dot(h, w2_ref[...],
                     preferred_element_type=jnp.float32) + b2_ref[...]

    idx = lax.broadcasted_iota(jnp.int32, (_B, _NE), 1)
    m1 = jnp.max(logits, axis=1, keepdims=True)
    i1 = jnp.min(jnp.where(logits == m1, idx, _NE), axis=1, keepdims=True)
    masked = jnp.where(idx == i1, -jnp.inf, logits)
    m2 = jnp.max(masked, axis=1, keepdims=True)
    i2 = jnp.min(jnp.where(masked == m2, idx, _NE), axis=1, keepdims=True)

    e = jnp.exp(m2 - m1)
    denom = 1.0 + e
    wout_ref[...] = jnp.concatenate([1.0 / denom, e / denom], axis=1)
    iout_ref[...] = jnp.concatenate([i1, i2], axis=1)


_SEL = np.kron(np.eye(_C, dtype=np.float32), np.ones((_L, 1), np.float32))


def kernel(x, W1, b1, W2, b2):
    partials = _sc_pool(x)
    sel = jnp.asarray(_SEL)
    b1r = b1.reshape(1, _MID)
    b2r = b2.reshape(1, _NE)

    wout, iout = pl.pallas_call(
        _router_tc_body,
        out_shape=[
            jax.ShapeDtypeStruct((_B, _K), jnp.float32),
            jax.ShapeDtypeStruct((_B, _K), jnp.int32),
        ],
    )(partials, sel, W1, b1r, W2, b2r)
    return wout, iout
